# baseline (speedup 1.0000x reference)
import jax
import jax.numpy as jnp
from jax.experimental import pallas as pl
from jax.experimental.pallas import tpu as pltpu

_NCHUNK = 4


def _body(x_hbm, cb_hbm, out_ref, x_v, cb_v, sem_x, sem_cb):
    M, D = x_v.shape
    K = cb_v.shape[0]
    KC = K // _NCHUNK
    cpx = pltpu.make_async_copy(x_hbm, x_v, sem_x)
    cpx.start()
    chunks = []
    for j in range(_NCHUNK):
        cp = pltpu.make_async_copy(cb_hbm.at[pl.ds(j * KC, KC)],
                                   cb_v.at[pl.ds(j * KC, KC)], sem_cb.at[j])
        cp.start()
        chunks.append(cp)
    cpx.wait()
    xv = x_v[...]
    x2 = jnp.sum(xv * xv, axis=1, keepdims=True)
    mins = None
    idx = None
    for j in range(_NCHUNK):
        chunks[j].wait()
        cbj = cb_v[pl.ds(j * KC, KC), :]
        c2 = jnp.sum(cbj * cbj, axis=1)[None, :]
        xc = jax.lax.dot_general(xv, cbj, (((1,), (1,)), ((), ())),
                                 preferred_element_type=jnp.float32)
        dist = x2 + c2 - 2.0 * xc
        mj = jnp.min(dist, axis=1, keepdims=True)
        kio = (jax.lax.broadcasted_iota(jnp.int32, (M, KC), 1)
               .astype(jnp.float32) + jnp.float32(j * KC))
        ij = jnp.min(jnp.where(dist == mj, kio, jnp.float32(K)),
                     axis=1, keepdims=True)
        if mins is None:
            mins, idx = mj, ij
        else:
            take_new = mj < mins
            mins = jnp.where(take_new, mj, mins)
            idx = jnp.where(take_new, ij, idx)
    kio_full = (jax.lax.broadcasted_iota(jnp.int32, (M, K), 1)
                .astype(jnp.float32))
    onehot = jnp.where(kio_full == idx, jnp.float32(1), jnp.float32(0))
    out_ref[...] = jax.lax.dot_general(
        onehot, cb_v[...], (((1,), (0,)), ((), ())),
        preferred_element_type=jnp.float32)


def kernel(x, codebook):
    N, C, H, W = x.shape
    K = codebook.shape[0]
    D = H * W
    M = N * C
    x_flat = x.reshape(M, D)
    cb_flat = codebook.reshape(K, D)
    out = pl.pallas_call(
        _body,
        in_specs=[pl.BlockSpec(memory_space=pl.ANY),
                  pl.BlockSpec(memory_space=pl.ANY)],
        out_shape=jax.ShapeDtypeStruct((M, D), jnp.float32),
        scratch_shapes=[
            pltpu.VMEM((M, D), jnp.float32),
            pltpu.VMEM((K, D), jnp.float32),
            pltpu.SemaphoreType.DMA,
            pltpu.SemaphoreType.DMA((_NCHUNK,)),
        ],
    )(x_flat, cb_flat)
    return out.reshape(N, C, H, W)


# speedup vs baseline: 1.0546x; 1.0546x over previous
import jax
import jax.numpy as jnp
from jax.experimental import pallas as pl
from jax.experimental.pallas import tpu as pltpu

_NCHUNK = 2


def _body(x_hbm, cb_hbm, out_ref, x_v, cb_v, sem_x, sem_cb):
    M, D = x_v.shape
    K = cb_v.shape[0]
    KC = K // _NCHUNK
    cpx = pltpu.make_async_copy(x_hbm, x_v, sem_x)
    cpx.start()
    chunks = []
    for j in range(_NCHUNK):
        cp = pltpu.make_async_copy(cb_hbm.at[pl.ds(j * KC, KC)],
                                   cb_v.at[pl.ds(j * KC, KC)], sem_cb.at[j])
        cp.start()
        chunks.append(cp)
    cpx.wait()
    xv = x_v[...]
    x2 = jnp.sum(xv * xv, axis=1, keepdims=True)
    mins = None
    idx = None
    for j in range(_NCHUNK):
        chunks[j].wait()
        cbj = cb_v[pl.ds(j * KC, KC), :]
        c2 = jnp.sum(cbj * cbj, axis=1)[None, :]
        xc = jax.lax.dot_general(xv, cbj, (((1,), (1,)), ((), ())),
                                 preferred_element_type=jnp.float32)
        dist = x2 + c2 - 2.0 * xc
        mj = jnp.min(dist, axis=1, keepdims=True)
        kio = (jax.lax.broadcasted_iota(jnp.int32, (M, KC), 1)
               .astype(jnp.float32) + jnp.float32(j * KC))
        ij = jnp.min(jnp.where(dist == mj, kio, jnp.float32(K)),
                     axis=1, keepdims=True)
        if mins is None:
            mins, idx = mj, ij
        else:
            take_new = mj < mins
            mins = jnp.where(take_new, mj, mins)
            idx = jnp.where(take_new, ij, idx)
    kio_full = (jax.lax.broadcasted_iota(jnp.int32, (M, K), 1)
                .astype(jnp.float32))
    onehot = jnp.where(kio_full == idx, jnp.float32(1), jnp.float32(0))
    out_ref[...] = jax.lax.dot_general(
        onehot, cb_v[...], (((1,), (0,)), ((), ())),
        preferred_element_type=jnp.float32)


def kernel(x, codebook):
    N, C, H, W = x.shape
    K = codebook.shape[0]
    D = H * W
    M = N * C
    x_flat = x.reshape(M, D)
    cb_flat = codebook.reshape(K, D)
    out = pl.pallas_call(
        _body,
        in_specs=[pl.BlockSpec(memory_space=pl.ANY),
                  pl.BlockSpec(memory_space=pl.ANY)],
        out_shape=jax.ShapeDtypeStruct((M, D), jnp.float32),
        scratch_shapes=[
            pltpu.VMEM((M, D), jnp.float32),
            pltpu.VMEM((K, D), jnp.float32),
            pltpu.SemaphoreType.DMA,
            pltpu.SemaphoreType.DMA((_NCHUNK,)),
        ],
    )(x_flat, cb_flat)
    return out.reshape(N, C, H, W)


# speedup vs baseline: 1.1248x; 1.0665x over previous
import jax
import jax.numpy as jnp
from jax.experimental import pallas as pl
from jax.experimental.pallas import tpu as pltpu


def _body(x_ref, cb_ref, out_hbm, out_v, sem_o):
    M, D = x_ref.shape
    K = cb_ref.shape[0]
    H = M // 2
    xv = x_ref[...]
    cb = cb_ref[...]
    x2 = jnp.sum(xv * xv, axis=1, keepdims=True)
    c2 = jnp.sum(cb * cb, axis=1)[None, :]
    xc = jax.lax.dot_general(xv, cb, (((1,), (1,)), ((), ())),
                             preferred_element_type=jnp.float32)
    dist = x2 + c2 - 2.0 * xc
    mins = jnp.min(dist, axis=1, keepdims=True)
    kio = jax.lax.broadcasted_iota(jnp.int32, (M, K), 1).astype(jnp.float32)
    idx = jnp.min(jnp.where(dist == mins, kio, jnp.float32(K)),
                  axis=1, keepdims=True)
    onehot = jnp.where(kio == idx, jnp.float32(1), jnp.float32(0))
    cps = []
    for h in range(2):
        rows = pl.ds(h * H, H)
        out_v[rows, :] = jax.lax.dot_general(
            onehot[h * H:(h + 1) * H, :], cb, (((1,), (0,)), ((), ())),
            preferred_element_type=jnp.float32)
        cp = pltpu.make_async_copy(out_v.at[rows], out_hbm.at[rows],
                                   sem_o.at[h])
        cp.start()
        cps.append(cp)
    for cp in cps:
        cp.wait()


def kernel(x, codebook):
    N, C, H, W = x.shape
    K = codebook.shape[0]
    D = H * W
    M = N * C
    x_flat = x.reshape(M, D)
    cb_flat = codebook.reshape(K, D)
    out = pl.pallas_call(
        _body,
        out_specs=pl.BlockSpec(memory_space=pl.ANY),
        out_shape=jax.ShapeDtypeStruct((M, D), jnp.float32),
        scratch_shapes=[
            pltpu.VMEM((M, D), jnp.float32),
            pltpu.SemaphoreType.DMA((2,)),
        ],
    )(x_flat, cb_flat)
    return out.reshape(N, C, H, W)


# speedup vs baseline: 1.1340x; 1.0082x over previous
import jax
import jax.numpy as jnp
from jax.experimental import pallas as pl
from jax.experimental.pallas import tpu as pltpu


def _body(x_ref, cb_ref, out_hbm, out_v, sem_o):
    M, D = x_ref.shape
    K = cb_ref.shape[0]
    H = M // 4
    xv = x_ref[...]
    cb = cb_ref[...]
    x2 = jnp.sum(xv * xv, axis=1, keepdims=True)
    c2 = jnp.sum(cb * cb, axis=1)[None, :]
    xc = jax.lax.dot_general(xv, cb, (((1,), (1,)), ((), ())),
                             preferred_element_type=jnp.float32)
    dist = x2 + c2 - 2.0 * xc
    mins = jnp.min(dist, axis=1, keepdims=True)
    kio = jax.lax.broadcasted_iota(jnp.int32, (M, K), 1).astype(jnp.float32)
    idx = jnp.min(jnp.where(dist == mins, kio, jnp.float32(K)),
                  axis=1, keepdims=True)
    onehot = jnp.where(kio == idx, jnp.float32(1), jnp.float32(0))
    cps = []
    for h in range(4):
        rows = pl.ds(h * H, H)
        out_v[rows, :] = jax.lax.dot_general(
            onehot[h * H:(h + 1) * H, :], cb, (((1,), (0,)), ((), ())),
            preferred_element_type=jnp.float32)
        cp = pltpu.make_async_copy(out_v.at[rows], out_hbm.at[rows],
                                   sem_o.at[h])
        cp.start()
        cps.append(cp)
    for cp in cps:
        cp.wait()


def kernel(x, codebook):
    N, C, H, W = x.shape
    K = codebook.shape[0]
    D = H * W
    M = N * C
    x_flat = x.reshape(M, D)
    cb_flat = codebook.reshape(K, D)
    out = pl.pallas_call(
        _body,
        out_specs=pl.BlockSpec(memory_space=pl.ANY),
        out_shape=jax.ShapeDtypeStruct((M, D), jnp.float32),
        scratch_shapes=[
            pltpu.VMEM((M, D), jnp.float32),
            pltpu.SemaphoreType.DMA((4,)),
        ],
    )(x_flat, cb_flat)
    return out.reshape(N, C, H, W)
